# baseline (speedup 1.0000x reference)
import functools

import jax
import jax.numpy as jnp
from jax import lax
from jax.experimental import pallas as pl
from jax.experimental.pallas import tpu as pltpu
from jax.experimental.pallas import tpu_sc as plsc

VOCAB = 1000000
EMBED_DIM = 64
IN_FEATURES = 128
OUT_FEATURES = 4
BATCH = 4096
SEQ = 200
TOKENS = 2 * SEQ
C0, C1 = 128, SEQ - 128
NC, NS = 2, 16
NW = NC * NS
B_PER_W = BATCH // NW
ROW_W = 80


def _sc_pool(table2, premise, hypothesis):
    mesh = plsc.VectorSubcoreMesh(core_axis_name="c", subcore_axis_name="s")

    @functools.partial(
        pl.kernel,
        mesh=mesh,
        compiler_params=pltpu.CompilerParams(use_tc_tiling_on_sc=False),
        out_type=jax.ShapeDtypeStruct((BATCH, EMBED_DIM), jnp.float32),
        scratch_types=[
            pltpu.VMEM((B_PER_W, SEQ), jnp.int32),
            pltpu.VMEM((B_PER_W, SEQ), jnp.int32),
            pltpu.VMEM((TOKENS, ROW_W), jnp.float32),
            pltpu.VMEM((TOKENS, ROW_W), jnp.float32),
            pltpu.VMEM((B_PER_W, EMBED_DIM), jnp.float32),
            pltpu.SemaphoreType.DMA,
            pltpu.SemaphoreType.DMA,
        ],
    )
    def pool(table_hbm, p_hbm, h_hbm, out_hbm,
             idxp_v, idxh_v, rows0_v, rows1_v, out_v, sem0, sem1):
        wid = lax.axis_index("s") * NC + lax.axis_index("c")
        base = wid * B_PER_W

        pltpu.sync_copy(p_hbm.at[pl.ds(base, B_PER_W)], idxp_v)
        pltpu.sync_copy(h_hbm.at[pl.ds(base, B_PER_W)], idxh_v)

        def fire(b, buf, sem):
            for iv, off in ((idxp_v, 0), (idxh_v, SEQ)):
                pltpu.async_copy(table_hbm.at[iv.at[b, pl.ds(0, C0)]],
                                 buf.at[pl.ds(off, C0)], sem)
                pltpu.async_copy(table_hbm.at[iv.at[b, pl.ds(C0, C1)]],
                                 buf.at[pl.ds(off + C0, C1)], sem)

        def drain(buf, sem):
            for off in (0, SEQ):
                pltpu.make_async_copy(table_hbm.at[idxp_v.at[0, pl.ds(0, C0)]],
                                      buf.at[pl.ds(off, C0)], sem).wait()
                pltpu.make_async_copy(table_hbm.at[idxp_v.at[0, pl.ds(C0, C1)]],
                                      buf.at[pl.ds(off + C0, C1)], sem).wait()

        def accumulate(b, buf):
            def add4(j, accs):
                a = list(accs)
                r = j * 4
                for k in range(4):
                    s = 4 * (k & 1)
                    a[s + 0] = a[s + 0] + buf[r + k, pl.ds(0, 16)]
                    a[s + 1] = a[s + 1] + buf[r + k, pl.ds(16, 16)]
                    a[s + 2] = a[s + 2] + buf[r + k, pl.ds(32, 16)]
                    a[s + 3] = a[s + 3] + buf[r + k, pl.ds(48, 16)]
                return tuple(a)

            z = jnp.zeros((16,), jnp.float32)
            a = lax.fori_loop(0, TOKENS // 4, add4, (z,) * 8)
            out_v[b, pl.ds(0, 16)] = a[0] + a[4]
            out_v[b, pl.ds(16, 16)] = a[1] + a[5]
            out_v[b, pl.ds(32, 16)] = a[2] + a[6]
            out_v[b, pl.ds(48, 16)] = a[3] + a[7]

        fire(0, rows0_v, sem0)

        def pair_body(g, carry):
            b0 = 2 * g
            fire(b0 + 1, rows1_v, sem1)
            drain(rows0_v, sem0)
            accumulate(b0, rows0_v)

            @pl.when(g < B_PER_W // 2 - 1)
            def _():
                fire(b0 + 2, rows0_v, sem0)

            drain(rows1_v, sem1)
            accumulate(b0 + 1, rows1_v)
            return carry

        lax.fori_loop(0, B_PER_W // 2, pair_body, 0)
        pltpu.sync_copy(out_v, out_hbm.at[pl.ds(base, B_PER_W)])

    return pool(table2, premise, hypothesis)


def _mlp_body(x_ref, w1_ref, b1_ref, w2_ref, b2_ref, o_ref):
    x = x_ref[...] * (1.0 / float(TOKENS))
    h = jnp.dot(x, w1_ref[...], preferred_element_type=jnp.float32)
    h = jnp.maximum(h + b1_ref[...], 0.0)
    o = jnp.dot(h, w2_ref[...], preferred_element_type=jnp.float32)
    o_ref[...] = o + b2_ref[...]


def _mlp(sums, W1, b1, W2, b2):
    return pl.pallas_call(
        _mlp_body,
        out_shape=jax.ShapeDtypeStruct((BATCH, OUT_FEATURES), jnp.float32),
    )(sums, W1, b1.reshape(1, IN_FEATURES), W2, b2.reshape(1, OUT_FEATURES))


@jax.jit
def kernel(premise, hypothesis, table, W1, b1, W2, b2):
    table2 = jnp.pad(table, ((0, 0), (0, ROW_W - EMBED_DIM)))
    sums = _sc_pool(table2, premise.astype(jnp.int32),
                    hypothesis.astype(jnp.int32))
    return _mlp(sums, W1, b1, W2, b2)


# speedup vs baseline: 1.8113x; 1.8113x over previous
import functools

import jax
import jax.numpy as jnp
from jax import lax
from jax.experimental import pallas as pl
from jax.experimental.pallas import tpu as pltpu
from jax.experimental.pallas import tpu_sc as plsc

VOCAB = 1000000
EMBED_DIM = 64
IN_FEATURES = 128
OUT_FEATURES = 4
BATCH = 4096
SEQ = 200
TOKENS = 2 * SEQ
C0, C1 = 128, SEQ - 128
NC, NS = 2, 16
NW = NC * NS
B_PER_W = BATCH // NW


def _sc_pool(table2, premise, hypothesis):
    mesh = plsc.VectorSubcoreMesh(core_axis_name="c", subcore_axis_name="s")

    @functools.partial(
        pl.kernel,
        mesh=mesh,
        compiler_params=pltpu.CompilerParams(use_tc_tiling_on_sc=False),
        out_type=jax.ShapeDtypeStruct((BATCH, EMBED_DIM), jnp.float32),
        scratch_types=[
            pltpu.VMEM((B_PER_W, SEQ), jnp.int32),
            pltpu.VMEM((B_PER_W, SEQ), jnp.int32),
            pltpu.VMEM((TOKENS, EMBED_DIM), jnp.float32),
            pltpu.VMEM((TOKENS, EMBED_DIM), jnp.float32),
            pltpu.VMEM((B_PER_W, EMBED_DIM), jnp.float32),
            pltpu.SemaphoreType.DMA,
            pltpu.SemaphoreType.DMA,
        ],
    )
    def pool(table_hbm, p_hbm, h_hbm, out_hbm,
             idxp_v, idxh_v, rows0_v, rows1_v, out_v, sem0, sem1):
        wid = lax.axis_index("s") * NC + lax.axis_index("c")
        base = wid * B_PER_W

        pltpu.sync_copy(p_hbm.at[pl.ds(base, B_PER_W)], idxp_v)
        pltpu.sync_copy(h_hbm.at[pl.ds(base, B_PER_W)], idxh_v)

        def fire(b, buf, sem):
            for iv, off in ((idxp_v, 0), (idxh_v, SEQ)):
                pltpu.async_copy(table_hbm.at[iv.at[b, pl.ds(0, C0)]],
                                 buf.at[pl.ds(off, C0)], sem)
                pltpu.async_copy(table_hbm.at[iv.at[b, pl.ds(C0, C1)]],
                                 buf.at[pl.ds(off + C0, C1)], sem)

        def drain(buf, sem):
            for off in (0, SEQ):
                pltpu.make_async_copy(table_hbm.at[idxp_v.at[0, pl.ds(0, C0)]],
                                      buf.at[pl.ds(off, C0)], sem).wait()
                pltpu.make_async_copy(table_hbm.at[idxp_v.at[0, pl.ds(C0, C1)]],
                                      buf.at[pl.ds(off + C0, C1)], sem).wait()

        def accumulate(b, buf):
            def add4(j, accs):
                a = list(accs)
                r = j * 8
                for k in range(8):
                    s = 4 * (k & 1)
                    a[s + 0] = a[s + 0] + buf[r + k, pl.ds(0, 16)]
                    a[s + 1] = a[s + 1] + buf[r + k, pl.ds(16, 16)]
                    a[s + 2] = a[s + 2] + buf[r + k, pl.ds(32, 16)]
                    a[s + 3] = a[s + 3] + buf[r + k, pl.ds(48, 16)]
                return tuple(a)

            z = jnp.zeros((16,), jnp.float32)
            a = lax.fori_loop(0, TOKENS // 8, add4, (z,) * 8)
            out_v[b, pl.ds(0, 16)] = a[0] + a[4]
            out_v[b, pl.ds(16, 16)] = a[1] + a[5]
            out_v[b, pl.ds(32, 16)] = a[2] + a[6]
            out_v[b, pl.ds(48, 16)] = a[3] + a[7]

        fire(0, rows0_v, sem0)

        def pair_body(g, carry):
            b0 = 2 * g
            fire(b0 + 1, rows1_v, sem1)
            drain(rows0_v, sem0)
            accumulate(b0, rows0_v)

            @pl.when(g < B_PER_W // 2 - 1)
            def _():
                fire(b0 + 2, rows0_v, sem0)

            drain(rows1_v, sem1)
            accumulate(b0 + 1, rows1_v)
            return carry

        lax.fori_loop(0, B_PER_W // 2, pair_body, 0)
        pltpu.sync_copy(out_v, out_hbm.at[pl.ds(base, B_PER_W)])

    return pool(table2, premise, hypothesis)


def _mlp_body(x_ref, w1_ref, b1_ref, w2_ref, b2_ref, o_ref):
    x = x_ref[...] * (1.0 / float(TOKENS))
    h = jnp.dot(x, w1_ref[...], preferred_element_type=jnp.float32)
    h = jnp.maximum(h + b1_ref[...], 0.0)
    o = jnp.dot(h, w2_ref[...], preferred_element_type=jnp.float32)
    o_ref[...] = o + b2_ref[...]


def _mlp(sums, W1, b1, W2, b2):
    return pl.pallas_call(
        _mlp_body,
        out_shape=jax.ShapeDtypeStruct((BATCH, OUT_FEATURES), jnp.float32),
    )(sums, W1, b1.reshape(1, IN_FEATURES), W2, b2.reshape(1, OUT_FEATURES))


@jax.jit
def kernel(premise, hypothesis, table, W1, b1, W2, b2):
    table2 = jnp.pad(table, ((0, 0), (0, EMBED_DIM))).reshape(2 * VOCAB, EMBED_DIM)
    p2 = premise.astype(jnp.int32) * 2
    h2 = hypothesis.astype(jnp.int32) * 2
    sums = _sc_pool(table2, p2, h2)
    return _mlp(sums, W1, b1, W2, b2)
